# Initial kernel scaffold; baseline (speedup 1.0000x reference)
import jax
import jax.numpy as jnp
from jax.experimental import pallas as pl


def kernel(species_ids, move_ids, item_ids, ability_ids, status_ids, hp_values, boost_values, mega_flags, species_table, move_table, item_table, ability_table, status_table, W1, b1, W2, b2):
    raise NotImplementedError("write your pallas kernel here")


# speedup vs baseline: 6.4442x; 6.4442x over previous
import functools

import jax
import jax.numpy as jnp
from jax import lax
from jax.experimental import pallas as pl
from jax.experimental.pallas import tpu as pltpu
from jax.experimental.pallas import tpu_sc as plsc


def _gelu(x):
    return 0.5 * x * (1.0 + lax.erf(x * 0.7071067811865476))


_CHUNK = 256


def _make_sc_gather(R):
    info = plsc.get_sparse_core_info()
    NW = info.num_cores * info.num_subcores
    NC = info.num_cores
    rows_per_w = R // NW
    nchunks = rows_per_w // _CHUNK
    G = _CHUNK // 128

    mesh = plsc.VectorSubcoreMesh(core_axis_name="c", subcore_axis_name="s")

    @functools.partial(
        pl.kernel,
        mesh=mesh,
        out_type=[jax.ShapeDtypeStruct((R, 128), jnp.float32),
                  jax.ShapeDtypeStruct((R, 128), jnp.float32)],
        scratch_types=(
            [pltpu.VMEM((rows_per_w,), jnp.int32) for _ in range(5)]
            + [pltpu.VMEM((_CHUNK, 128), jnp.float32) for _ in range(2)]
            + [pltpu.SemaphoreType.DMA, pltpu.SemaphoreType.DMA]
        ),
    )
    def sc_kernel(sp_idx, m0_idx, m1_idx, m2_idx, m3_idx,
                  spP, mv0P, mv1P, mv2P, mv3P,
                  hA, hB, sp_v, m0_v, m1_v, m2_v, m3_v,
                  hA_v, hB_v, gsem, asem):
        wid = lax.axis_index("s") * NC + lax.axis_index("c")
        row_base = pl.multiple_of(wid * rows_per_w, rows_per_w)

        for idx_hbm, idx_v in ((sp_idx, sp_v), (m0_idx, m0_v),
                               (m1_idx, m1_v), (m2_idx, m2_v),
                               (m3_idx, m3_v)):
            pltpu.sync_copy(idx_hbm.at[pl.ds(row_base, rows_per_w)], idx_v)

        def body(c, carry):
            off = c * _CHUNK
            row0 = row_base + off
            base = [pltpu.async_copy(
                        spP.at[sp_v.at[pl.ds(off + j * 128, 128)]],
                        hA_v.at[pl.ds(j * 128, 128)], gsem)
                    for j in range(G)]
            base += [pltpu.async_copy(
                         mv2P.at[m2_v.at[pl.ds(off + j * 128, 128)]],
                         hB_v.at[pl.ds(j * 128, 128)], gsem)
                     for j in range(G)]
            for cp in base:
                cp.wait()
            acc = []
            for tab, idx_v, dst in ((mv0P, m0_v, hA_v), (mv1P, m1_v, hA_v),
                                    (mv3P, m3_v, hB_v)):
                acc += [pltpu.async_copy(
                            tab.at[idx_v.at[pl.ds(off + j * 128, 128)]],
                            dst.at[pl.ds(j * 128, 128)], asem, add=True)
                        for j in range(G)]
            for cp in acc:
                cp.wait()
            ws = [pltpu.async_copy(hA_v, hA.at[pl.ds(row0, _CHUNK)], gsem),
                  pltpu.async_copy(hB_v, hB.at[pl.ds(row0, _CHUNK)], gsem)]
            for w in ws:
                w.wait()
            return carry

        lax.fori_loop(0, nchunks, body, 0)

    return sc_kernel


def _mlp_body(hA_ref, hB_ref, it_ref, ab_ref, st_ref, hp_ref, bo_ref, mg_ref,
              itT_ref, abT_ref, stT_ref, w1ab_ref, w1t_ref, b1_ref,
              w2_ref, b2_ref, o_ref):
    blk = hA_ref.shape[0]

    def onehot_emb(ids_ref, tab_ref):
        v = tab_ref.shape[0]
        ids = ids_ref[0, 0, :]
        oh = (ids[:, None] == lax.broadcasted_iota(jnp.int32, (blk, v), 1))
        return jnp.dot(oh.astype(jnp.bfloat16), tab_ref[...],
                       preferred_element_type=jnp.float32)

    tail = jnp.concatenate(
        [onehot_emb(it_ref, itT_ref), onehot_emb(ab_ref, abT_ref),
         onehot_emb(st_ref, stT_ref),
         hp_ref[...], bo_ref[...], mg_ref[...]], axis=1)
    xab = jnp.concatenate([hA_ref[...], hB_ref[...]], axis=1)
    h = (jnp.dot(xab, w1ab_ref[...], preferred_element_type=jnp.float32)
         + jnp.dot(tail, w1t_ref[...], preferred_element_type=jnp.float32))
    h = _gelu(h + b1_ref[...])
    o = jnp.dot(h, w2_ref[...], preferred_element_type=jnp.float32)
    o_ref[...] = _gelu(o + b2_ref[...])


def kernel(species_ids, move_ids, item_ids, ability_ids, status_ids,
           hp_values, boost_values, mega_flags, species_table, move_table,
           item_table, ability_table, status_table, W1, b1, W2, b2):
    B, N = species_ids.shape
    R = B * N
    IN_D, HIDDEN = W1.shape
    OUT_D = W2.shape[1]
    SP_D = species_table.shape[1]
    MV_D = move_table.shape[1]
    IT_D = item_table.shape[1]
    AB_D = ability_table.shape[1]
    ST_D = status_table.shape[1]

    def flat(ids):
        return ids.reshape(R).astype(jnp.int32)

    def pad_band(tab, lo, width=128):
        v, d = tab.shape
        return jnp.concatenate(
            [jnp.zeros((v, lo), jnp.float32), tab,
             jnp.zeros((v, width - lo - d), jnp.float32)], axis=1)

    spP = pad_band(species_table, 0)
    mv0P = pad_band(move_table, SP_D)
    mv1P = pad_band(move_table, SP_D + MV_D)
    mv2P = pad_band(move_table, 0)
    mv3P = pad_band(move_table, MV_D)

    hA, hB = _make_sc_gather(R)(
        flat(species_ids), flat(move_ids[..., 0]), flat(move_ids[..., 1]),
        flat(move_ids[..., 2]), flat(move_ids[..., 3]),
        spP, mv0P, mv1P, mv2P, mv3P)

    W1ab = jnp.concatenate(
        [W1[:SP_D + 2 * MV_D], W1[SP_D + 2 * MV_D:SP_D + 4 * MV_D],
         jnp.zeros((64, HIDDEN), jnp.float32)], axis=0)
    W1tail = W1[SP_D + 4 * MV_D:]

    BLK = 512
    GRD = R // BLK
    ids3 = lambda ids: flat(ids).reshape(GRD, 1, BLK)

    out = pl.pallas_call(
        _mlp_body,
        grid=(GRD,),
        in_specs=[
            pl.BlockSpec((BLK, 128), lambda i: (i, 0)),
            pl.BlockSpec((BLK, 128), lambda i: (i, 0)),
            pl.BlockSpec((1, 1, BLK), lambda i: (i, 0, 0)),
            pl.BlockSpec((1, 1, BLK), lambda i: (i, 0, 0)),
            pl.BlockSpec((1, 1, BLK), lambda i: (i, 0, 0)),
            pl.BlockSpec((BLK, 1), lambda i: (i, 0)),
            pl.BlockSpec((BLK, 6), lambda i: (i, 0)),
            pl.BlockSpec((BLK, 1), lambda i: (i, 0)),
            pl.BlockSpec(item_table.shape, lambda i: (0, 0)),
            pl.BlockSpec(ability_table.shape, lambda i: (0, 0)),
            pl.BlockSpec(status_table.shape, lambda i: (0, 0)),
            pl.BlockSpec((IN_D, HIDDEN), lambda i: (0, 0)),
            pl.BlockSpec((IT_D + AB_D + ST_D + 8, HIDDEN), lambda i: (0, 0)),
            pl.BlockSpec((1, HIDDEN), lambda i: (0, 0)),
            pl.BlockSpec((HIDDEN, OUT_D), lambda i: (0, 0)),
            pl.BlockSpec((1, OUT_D), lambda i: (0, 0)),
        ],
        out_specs=pl.BlockSpec((BLK, OUT_D), lambda i: (i, 0)),
        out_shape=jax.ShapeDtypeStruct((R, OUT_D), jnp.float32),
    )(hA, hB, ids3(item_ids), ids3(ability_ids), ids3(status_ids),
      hp_values.reshape(R, 1), boost_values.reshape(R, 6),
      mega_flags.reshape(R, 1),
      item_table.astype(jnp.bfloat16), ability_table.astype(jnp.bfloat16),
      status_table.astype(jnp.bfloat16),
      W1ab, W1tail, b1.reshape(1, HIDDEN), W2, b2.reshape(1, OUT_D))

    return out.reshape(B, N, OUT_D)


# speedup vs baseline: 7.4450x; 1.1553x over previous
import functools

import jax
import jax.numpy as jnp
from jax import lax
from jax.experimental import pallas as pl
from jax.experimental.pallas import tpu as pltpu
from jax.experimental.pallas import tpu_sc as plsc


def _gelu(x):
    return 0.5 * x * (1.0 + lax.erf(x * 0.7071067811865476))


_CHUNK = 256


def _make_sc_gather(R):
    info = plsc.get_sparse_core_info()
    NW = info.num_cores * info.num_subcores
    NC = info.num_cores
    rows_per_w = R // NW
    nchunks = rows_per_w // _CHUNK
    G = _CHUNK // 128

    mesh = plsc.VectorSubcoreMesh(core_axis_name="c", subcore_axis_name="s")

    @functools.partial(
        pl.kernel,
        mesh=mesh,
        out_type=[jax.ShapeDtypeStruct((R, 128), jnp.float32),
                  jax.ShapeDtypeStruct((R, 128), jnp.float32)],
        scratch_types=(
            [pltpu.VMEM((rows_per_w,), jnp.int32) for _ in range(5)]
            + [pltpu.VMEM((_CHUNK, 128), jnp.float32) for _ in range(2)]
            + [pltpu.SemaphoreType.DMA, pltpu.SemaphoreType.DMA]
        ),
    )
    def sc_kernel(sp_idx, m0_idx, m1_idx, m2_idx, m3_idx,
                  spP, mv0P, mv1P, mv2P, mv3P,
                  hA, hB, sp_v, m0_v, m1_v, m2_v, m3_v,
                  hA_v, hB_v, gsem, asem):
        wid = lax.axis_index("s") * NC + lax.axis_index("c")
        row_base = pl.multiple_of(wid * rows_per_w, rows_per_w)

        for idx_hbm, idx_v in ((sp_idx, sp_v), (m0_idx, m0_v),
                               (m1_idx, m1_v), (m2_idx, m2_v),
                               (m3_idx, m3_v)):
            pltpu.sync_copy(idx_hbm.at[pl.ds(row_base, rows_per_w)], idx_v)

        def body(c, carry):
            off = c * _CHUNK
            row0 = row_base + off
            base = [pltpu.async_copy(
                        spP.at[sp_v.at[pl.ds(off + j * 128, 128)]],
                        hA_v.at[pl.ds(j * 128, 128)], gsem)
                    for j in range(G)]
            base += [pltpu.async_copy(
                         mv2P.at[m2_v.at[pl.ds(off + j * 128, 128)]],
                         hB_v.at[pl.ds(j * 128, 128)], gsem)
                     for j in range(G)]
            for cp in base:
                cp.wait()
            acc = []
            for tab, idx_v, dst in ((mv0P, m0_v, hA_v), (mv1P, m1_v, hA_v),
                                    (mv3P, m3_v, hB_v)):
                acc += [pltpu.async_copy(
                            tab.at[idx_v.at[pl.ds(off + j * 128, 128)]],
                            dst.at[pl.ds(j * 128, 128)], asem, add=True)
                        for j in range(G)]
            for cp in acc:
                cp.wait()
            ws = [pltpu.async_copy(hA_v, hA.at[pl.ds(row0, _CHUNK)], gsem),
                  pltpu.async_copy(hB_v, hB.at[pl.ds(row0, _CHUNK)], gsem)]
            for w in ws:
                w.wait()
            return carry

        lax.fori_loop(0, nchunks, body, 0)

    return sc_kernel


def _mlp_body(hA_ref, hB_ref, it_ref, ab_ref, st_ref, hp_ref, bo_ref, mg_ref,
              itT_ref, abT_ref, stT_ref, w1ab_ref, w1t_ref, b1_ref,
              w2_ref, b2_ref, o_ref):
    blk = hA_ref.shape[0]

    def onehot_emb(ids_ref, tab_ref):
        v = tab_ref.shape[0]
        ids = ids_ref[0, 0, :]
        oh = (ids[:, None] == lax.broadcasted_iota(jnp.int32, (blk, v), 1))
        return jnp.dot(oh.astype(jnp.bfloat16), tab_ref[...],
                       preferred_element_type=jnp.float32)

    tail = jnp.concatenate(
        [onehot_emb(it_ref, itT_ref), onehot_emb(ab_ref, abT_ref),
         onehot_emb(st_ref, stT_ref),
         hp_ref[...], bo_ref[...], mg_ref[...]], axis=1)
    xab = jnp.concatenate([hA_ref[...], hB_ref[...]], axis=1)
    h = (jnp.dot(xab.astype(jnp.bfloat16), w1ab_ref[...],
                 preferred_element_type=jnp.float32)
         + jnp.dot(tail.astype(jnp.bfloat16), w1t_ref[...],
                   preferred_element_type=jnp.float32))
    h = _gelu(h + b1_ref[...])
    o = jnp.dot(h.astype(jnp.bfloat16), w2_ref[...],
                preferred_element_type=jnp.float32)
    o_ref[...] = _gelu(o + b2_ref[...])


def kernel(species_ids, move_ids, item_ids, ability_ids, status_ids,
           hp_values, boost_values, mega_flags, species_table, move_table,
           item_table, ability_table, status_table, W1, b1, W2, b2):
    B, N = species_ids.shape
    R = B * N
    IN_D, HIDDEN = W1.shape
    OUT_D = W2.shape[1]
    SP_D = species_table.shape[1]
    MV_D = move_table.shape[1]
    IT_D = item_table.shape[1]
    AB_D = ability_table.shape[1]
    ST_D = status_table.shape[1]

    def flat(ids):
        return ids.reshape(R).astype(jnp.int32)

    mvT = jnp.moveaxis(move_ids, -1, 0).reshape(move_ids.shape[-1], R)
    mvT = mvT.astype(jnp.int32)

    def pad_band(tab, lo, width=128):
        v, d = tab.shape
        return jnp.concatenate(
            [jnp.zeros((v, lo), jnp.float32), tab,
             jnp.zeros((v, width - lo - d), jnp.float32)], axis=1)

    spP = pad_band(species_table, 0)
    mv0P = pad_band(move_table, SP_D)
    mv1P = pad_band(move_table, SP_D + MV_D)
    mv2P = pad_band(move_table, 0)
    mv3P = pad_band(move_table, MV_D)

    hA, hB = _make_sc_gather(R)(
        flat(species_ids), mvT[0], mvT[1], mvT[2], mvT[3],
        spP, mv0P, mv1P, mv2P, mv3P)

    W1ab = jnp.concatenate(
        [W1[:SP_D + 2 * MV_D], W1[SP_D + 2 * MV_D:SP_D + 4 * MV_D],
         jnp.zeros((64, HIDDEN), jnp.float32)], axis=0)
    W1tail = W1[SP_D + 4 * MV_D:]

    BLK = 1024
    GRD = R // BLK
    ids3 = lambda ids: flat(ids).reshape(GRD, 1, BLK)

    out = pl.pallas_call(
        _mlp_body,
        grid=(GRD,),
        in_specs=[
            pl.BlockSpec((BLK, 128), lambda i: (i, 0)),
            pl.BlockSpec((BLK, 128), lambda i: (i, 0)),
            pl.BlockSpec((1, 1, BLK), lambda i: (i, 0, 0)),
            pl.BlockSpec((1, 1, BLK), lambda i: (i, 0, 0)),
            pl.BlockSpec((1, 1, BLK), lambda i: (i, 0, 0)),
            pl.BlockSpec((BLK, 1), lambda i: (i, 0)),
            pl.BlockSpec((BLK, 6), lambda i: (i, 0)),
            pl.BlockSpec((BLK, 1), lambda i: (i, 0)),
            pl.BlockSpec(item_table.shape, lambda i: (0, 0)),
            pl.BlockSpec(ability_table.shape, lambda i: (0, 0)),
            pl.BlockSpec(status_table.shape, lambda i: (0, 0)),
            pl.BlockSpec((IN_D, HIDDEN), lambda i: (0, 0)),
            pl.BlockSpec((IT_D + AB_D + ST_D + 8, HIDDEN), lambda i: (0, 0)),
            pl.BlockSpec((1, HIDDEN), lambda i: (0, 0)),
            pl.BlockSpec((HIDDEN, OUT_D), lambda i: (0, 0)),
            pl.BlockSpec((1, OUT_D), lambda i: (0, 0)),
        ],
        out_specs=pl.BlockSpec((BLK, OUT_D), lambda i: (i, 0)),
        out_shape=jax.ShapeDtypeStruct((R, OUT_D), jnp.float32),
    )(hA, hB, ids3(item_ids), ids3(ability_ids), ids3(status_ids),
      hp_values.reshape(R, 1), boost_values.reshape(R, 6),
      mega_flags.reshape(R, 1),
      item_table.astype(jnp.bfloat16), ability_table.astype(jnp.bfloat16),
      status_table.astype(jnp.bfloat16),
      W1ab.astype(jnp.bfloat16), W1tail.astype(jnp.bfloat16),
      b1.reshape(1, HIDDEN), W2.astype(jnp.bfloat16),
      b2.reshape(1, OUT_D))

    return out.reshape(B, N, OUT_D)
